# Initial kernel scaffold; baseline (speedup 1.0000x reference)
import jax
import jax.numpy as jnp
from jax.experimental import pallas as pl


def kernel(x, theta, bias, W_cls, b_cls):
    raise NotImplementedError("write your pallas kernel here")


# speedup vs baseline: 4.0726x; 4.0726x over previous
import functools

import jax
import jax.numpy as jnp
from jax.experimental import pallas as pl

_K = 3
_TR = 256


def _knn_body(xi_ref, xt_ref, aai_ref, aaj_ref, iout_ref, wout_ref, dout_ref, *, n):
    xi = xi_ref[0]
    xt = xt_ref[0]
    aa_i = aai_ref[0, 0]
    aa_j = aaj_ref[0, 0]
    dotp = jnp.dot(xi, xt, preferred_element_type=jnp.float32)
    d2 = aa_i[:, None] + aa_j[None, :] - 2.0 * dotp
    d2 = jnp.maximum(d2, 0.0)
    dis = jnp.sqrt(d2)
    avg = jnp.mean(dis, axis=1)
    denom = avg * avg + 1e-12
    iota = jax.lax.broadcasted_iota(jnp.int32, (_TR, n), 1)
    cur = dis
    ws = []
    for k in range(_K):
        m = jnp.min(cur, axis=1)
        eq = cur == m[:, None]
        ik = jnp.min(jnp.where(eq, iota, n), axis=1)
        wk = jnp.exp(-(m * m) / denom)
        iout_ref[0, k, :] = ik
        wout_ref[0, k, :] = wk
        ws.append(wk)
        if k < _K - 1:
            cur = jnp.where(iota == ik[:, None], jnp.inf, cur)
    dout_ref[0, 0, :] = 1.0 / (ws[0] + ws[1] + ws[2])


def _knn_topk(xl):
    b, n, _ = xl.shape
    xp = jnp.pad(xl, ((0, 0), (0, 0), (0, 5)))
    xpt = xp.transpose(0, 2, 1)
    aa = jnp.sum(xl * xl, axis=2).reshape(b, 1, n)
    nb = n // _TR
    grid = (b, nb)
    return pl.pallas_call(
        functools.partial(_knn_body, n=n),
        grid=grid,
        in_specs=[
            pl.BlockSpec((1, _TR, 8), lambda bi, ri: (bi, ri, 0)),
            pl.BlockSpec((1, 8, n), lambda bi, ri: (bi, 0, 0)),
            pl.BlockSpec((1, 1, _TR), lambda bi, ri: (bi, 0, ri)),
            pl.BlockSpec((1, 1, n), lambda bi, ri: (bi, 0, 0)),
        ],
        out_specs=[
            pl.BlockSpec((1, _K, _TR), lambda bi, ri: (bi, 0, ri)),
            pl.BlockSpec((1, _K, _TR), lambda bi, ri: (bi, 0, ri)),
            pl.BlockSpec((1, 1, _TR), lambda bi, ri: (bi, 0, ri)),
        ],
        out_shape=[
            jax.ShapeDtypeStruct((b, _K, n), jnp.int32),
            jax.ShapeDtypeStruct((b, _K, n), jnp.float32),
            jax.ShapeDtypeStruct((b, 1, n), jnp.float32),
        ],
    )(xp, xpt, aa, aa)


def kernel(x, theta, bias, W_cls, b_cls):
    b, _, n, _ = x.shape
    hid = theta.shape[1]
    e = b * n
    xl = x[:, -1]
    idx, w, invde = _knn_topk(xl)
    gidx = (idx + (jnp.arange(b, dtype=jnp.int32) * n)[:, None, None])
    gidx = gidx.transpose(0, 2, 1).reshape(e, _K)
    wf = w.transpose(0, 2, 1).reshape(e, _K)
    invdef = invde.reshape(e)

    dv = jnp.zeros((e,), jnp.float32).at[gidx.reshape(-1)].add(wf.reshape(-1))
    dv2 = jnp.where(dv > 0, dv ** -0.5, 0.0)

    theta_t = jnp.broadcast_to(theta[None], (b, n, hid)).reshape(e, hid)

    def apply_op(u):
        t = jnp.sum(wf[:, :, None] * u[gidx], axis=1)
        s = invdef[:, None] * t
        p = (wf[:, :, None] * s[:, None, :]).reshape(-1, hid)
        y = jnp.zeros((e, hid), jnp.float32).at[gidx.reshape(-1)].add(p)
        return dv2[:, None] * y

    h1 = apply_op(dv2[:, None] * theta_t) + bias
    h2 = apply_op(dv2[:, None] * h1)
    f = jnp.max(h2.reshape(b, n, hid), axis=1)
    return f @ W_cls.T + b_cls


# speedup vs baseline: 10.5151x; 2.5819x over previous
import functools

import jax
import jax.numpy as jnp
from jax import lax
from jax.experimental import pallas as pl
from jax.experimental.pallas import tpu as pltpu
from jax.experimental.pallas import tpu_sc as plsc

_K = 3
_TR = 256
_NW = 32
_NSUB = 16
_CH = 128


def _knn_body(xi_ref, xt_ref, aai_ref, aaj_ref, iout_ref, wout_ref, dout_ref, *, n):
    xi = xi_ref[0]
    xt = xt_ref[0]
    aa_i = aai_ref[0, 0]
    aa_j = aaj_ref[0, 0]
    dotp = jnp.dot(xi, xt, preferred_element_type=jnp.float32)
    d2 = aa_i[:, None] + aa_j[None, :] - 2.0 * dotp
    d2 = jnp.maximum(d2, 0.0)
    dis = jnp.sqrt(d2)
    avg = jnp.mean(dis, axis=1)
    denom = avg * avg + 1e-12
    iota = jax.lax.broadcasted_iota(jnp.int32, (_TR, n), 1)
    cur = dis
    ws = []
    for k in range(_K):
        m = jnp.min(cur, axis=1)
        eq = cur == m[:, None]
        ik = jnp.min(jnp.where(eq, iota, n), axis=1)
        wk = jnp.exp(-(m * m) / denom)
        iout_ref[0, k, :] = ik
        wout_ref[0, k, :] = wk
        ws.append(wk)
        if k < _K - 1:
            cur = jnp.where(iota == ik[:, None], jnp.inf, cur)
    dout_ref[0, 0, :] = 1.0 / (ws[0] + ws[1] + ws[2])


def _knn_topk(xl):
    b, n, _ = xl.shape
    xp = jnp.pad(xl, ((0, 0), (0, 0), (0, 5)))
    xpt = xp.transpose(0, 2, 1)
    aa = jnp.sum(xl * xl, axis=2).reshape(b, 1, n)
    grid = (b, n // _TR)
    return pl.pallas_call(
        functools.partial(_knn_body, n=n),
        grid=grid,
        in_specs=[
            pl.BlockSpec((1, _TR, 8), lambda bi, ri: (bi, ri, 0)),
            pl.BlockSpec((1, 8, n), lambda bi, ri: (bi, 0, 0)),
            pl.BlockSpec((1, 1, _TR), lambda bi, ri: (bi, 0, ri)),
            pl.BlockSpec((1, 1, n), lambda bi, ri: (bi, 0, 0)),
        ],
        out_specs=[
            pl.BlockSpec((1, _K, _TR), lambda bi, ri: (bi, 0, ri)),
            pl.BlockSpec((1, _K, _TR), lambda bi, ri: (bi, 0, ri)),
            pl.BlockSpec((1, 1, _TR), lambda bi, ri: (bi, 0, ri)),
        ],
        out_shape=[
            jax.ShapeDtypeStruct((b, _K, n), jnp.int32),
            jax.ShapeDtypeStruct((b, _K, n), jnp.float32),
            jax.ShapeDtypeStruct((b, 1, n), jnp.float32),
        ],
    )(xp, xpt, aa, aa)


def _sc_mesh():
    return plsc.VectorSubcoreMesh(core_axis_name="c", subcore_axis_name="s")


def _sc_gather(table, idx3):
    e, d = table.shape
    nch = idx3.shape[1]
    rw = nch * _CH
    r = _NW * rw

    @functools.partial(
        pl.kernel,
        out_type=jax.ShapeDtypeStruct((r, d), jnp.float32),
        mesh=_sc_mesh(),
        scratch_types=[
            pltpu.VMEM((nch, _CH), jnp.int32),
            pltpu.VMEM((rw, d), jnp.float32),
            pltpu.SemaphoreType.DMA,
        ],
        compiler_params=pltpu.CompilerParams(use_tc_tiling_on_sc=False),
    )
    def k(table_hbm, idx_hbm, out_hbm, idx_v, rows_v, sem):
        wid = lax.axis_index("s") * 2 + lax.axis_index("c")
        pltpu.sync_copy(idx_hbm.at[wid], idx_v)
        descs = []
        for c in range(nch):
            descs.append(pltpu.async_copy(
                table_hbm.at[idx_v.at[c]], rows_v.at[pl.ds(c * _CH, _CH)], sem))
        for de in descs:
            de.wait()
        pltpu.sync_copy(rows_v, out_hbm.at[pl.ds(wid * rw, rw)])

    return k(table, idx3)


def _sc_scatter_add(rows, idx3, e):
    r, d = rows.shape
    nch = idx3.shape[1]
    rw = nch * _CH
    npers = e // _NSUB
    zer = jnp.zeros((e, d), jnp.float32)

    @functools.partial(
        pl.kernel,
        out_type=jax.ShapeDtypeStruct((2, e, d), jnp.float32),
        mesh=_sc_mesh(),
        scratch_types=[
            pltpu.VMEM((nch, _CH), jnp.int32),
            pltpu.VMEM((rw, d), jnp.float32),
            pltpu.VMEM_SHARED((e, d), jnp.float32),
        ],
        compiler_params=pltpu.CompilerParams(use_tc_tiling_on_sc=False),
    )
    def k(rows_hbm, idx_hbm, zer_hbm, out_hbm, idx_v, rows_v, shared):
        cid = lax.axis_index("c")
        sid = lax.axis_index("s")
        wid = sid * 2 + cid
        pltpu.sync_copy(zer_hbm.at[pl.ds(sid * npers, npers)],
                        shared.at[pl.ds(sid * npers, npers)])
        pltpu.sync_copy(idx_hbm.at[wid], idx_v)
        pltpu.sync_copy(rows_hbm.at[pl.ds(wid * rw, rw)], rows_v)
        plsc.subcore_barrier()
        for c in range(nch):
            pltpu.sync_copy(rows_v.at[pl.ds(c * _CH, _CH)],
                            shared.at[idx_v.at[c]], add=True)
        plsc.subcore_barrier()
        pltpu.sync_copy(shared.at[pl.ds(sid * npers, npers)],
                        out_hbm.at[cid, pl.ds(sid * npers, npers)])

    return k(rows, idx3, zer)


def _combine_body(g_ref, w_ref, inv_ref, p_ref):
    t = (g_ref[0] * w_ref[0, 0][:, None]
         + g_ref[1] * w_ref[1, 0][:, None]
         + g_ref[2] * w_ref[2, 0][:, None])
    s = inv_ref[0, 0][:, None] * t
    for k in range(_K):
        p_ref[k] = w_ref[k, 0][:, None] * s


def _tc_combine(g, w3, invde3, e, d):
    te = 2048
    return pl.pallas_call(
        _combine_body,
        grid=(e // te,),
        in_specs=[
            pl.BlockSpec((_K, te, d), lambda i: (0, i, 0)),
            pl.BlockSpec((_K, 1, te), lambda i: (0, 0, i)),
            pl.BlockSpec((1, 1, te), lambda i: (0, 0, i)),
        ],
        out_specs=pl.BlockSpec((_K, te, d), lambda i: (0, i, 0)),
        out_shape=jax.ShapeDtypeStruct((_K, e, d), jnp.float32),
    )(g, w3, invde3)


def _final_body(y_ref, dv2_ref, wc_ref, bc_ref, out_ref):
    y = y_ref[0] + y_ref[1]
    h2 = dv2_ref[0, 0][:, None] * y
    f = jnp.max(h2, axis=0)
    out_ref[0, 0] = jnp.sum(wc_ref[...] * f[None, :], axis=1) + bc_ref[0]


def _tc_final(y, dv2_3, W_cls, b_cls, b, n, d):
    cls = W_cls.shape[0]
    out = pl.pallas_call(
        _final_body,
        grid=(b,),
        in_specs=[
            pl.BlockSpec((2, n, d), lambda bi: (0, bi, 0)),
            pl.BlockSpec((1, 1, n), lambda bi: (0, 0, bi)),
            pl.BlockSpec((cls, d), lambda bi: (0, 0)),
            pl.BlockSpec((1, cls), lambda bi: (0, 0)),
        ],
        out_specs=pl.BlockSpec((1, 1, cls), lambda bi: (bi, 0, 0)),
        out_shape=jax.ShapeDtypeStruct((b, 1, cls), jnp.float32),
    )(y, dv2_3, W_cls, b_cls.reshape(1, cls))
    return out.reshape(b, cls)


def kernel(x, theta, bias, W_cls, b_cls):
    b, _, n, _ = x.shape
    hid = theta.shape[1]
    e = b * n
    r = e * _K
    nch = r // (_NW * _CH)
    xl = x[:, -1]
    idx, w, invde = _knn_topk(xl)

    gidx = (idx + (jnp.arange(b, dtype=jnp.int32) * n)[:, None, None])
    gidx_t = gidx.transpose(1, 0, 2).reshape(_K, e)
    idx3 = gidx_t.reshape(_NW, nch, _CH)
    wt = w.transpose(1, 0, 2).reshape(_K, e)
    w3 = wt.reshape(_K, 1, e)
    invde3 = invde.reshape(1, 1, e)

    p_dv = jnp.broadcast_to(wt.reshape(r)[:, None], (r, 16))
    dv_part = _sc_scatter_add(p_dv, idx3, e)
    dv = dv_part[0, :, 0] + dv_part[1, :, 0]
    dv2 = jnp.where(dv > 0, dv ** -0.5, 0.0)

    theta_t = jnp.broadcast_to(theta[None], (b, n, hid)).reshape(e, hid)

    def apply_op(u):
        g = _sc_gather(u, idx3).reshape(_K, e, hid)
        p = _tc_combine(g, w3, invde3, e, hid).reshape(r, hid)
        return _sc_scatter_add(p, idx3, e)

    y0 = apply_op(dv2[:, None] * theta_t)
    h1 = dv2[:, None] * (y0[0] + y0[1]) + bias
    y1 = apply_op(dv2[:, None] * h1)
    return _tc_final(y1, dv2.reshape(1, 1, e), W_cls, b_cls, b, n, hid)


# speedup vs baseline: 10.7782x; 1.0250x over previous
import functools

import jax
import jax.numpy as jnp
from jax import lax
from jax.experimental import pallas as pl
from jax.experimental.pallas import tpu as pltpu
from jax.experimental.pallas import tpu_sc as plsc

_K = 3
_TR = 256
_NW = 32
_NSUB = 16
_CH = 128


def _knn_body(xi_ref, xt_ref, aai_ref, aaj_ref, iout_ref, wout_ref, dout_ref, *, n):
    xi = xi_ref[0]
    xt = xt_ref[0]
    aa_i = aai_ref[0, 0]
    aa_j = aaj_ref[0, 0]
    dotp = jnp.dot(xi, xt, preferred_element_type=jnp.float32)
    d2 = aa_i[:, None] + aa_j[None, :] - 2.0 * dotp
    d2 = jnp.maximum(d2, 0.0)
    dis = jnp.sqrt(d2)
    avg = jnp.mean(dis, axis=1)
    denom = avg * avg + 1e-12
    iota = jax.lax.broadcasted_iota(jnp.int32, (_TR, n), 1).astype(jnp.float32)
    cur = dis
    ws = []
    for k in range(_K):
        m = jnp.min(cur, axis=1)
        eq = cur == m[:, None]
        ikf = jnp.min(jnp.where(eq, iota, jnp.float32(n)), axis=1)
        wk = jnp.exp(-(m * m) / denom)
        iout_ref[0, k, :] = ikf.astype(jnp.int32)
        wout_ref[0, k, :] = wk
        ws.append(wk)
        if k < _K - 1:
            cur = jnp.where(iota == ikf[:, None], jnp.inf, cur)
    dout_ref[0, 0, :] = 1.0 / (ws[0] + ws[1] + ws[2])


def _knn_topk(xl):
    b, n, _ = xl.shape
    xp = jnp.pad(xl, ((0, 0), (0, 0), (0, 5)))
    xpt = xp.transpose(0, 2, 1)
    aa = jnp.sum(xl * xl, axis=2).reshape(b, 1, n)
    grid = (b, n // _TR)
    return pl.pallas_call(
        functools.partial(_knn_body, n=n),
        grid=grid,
        in_specs=[
            pl.BlockSpec((1, _TR, 8), lambda bi, ri: (bi, ri, 0)),
            pl.BlockSpec((1, 8, n), lambda bi, ri: (bi, 0, 0)),
            pl.BlockSpec((1, 1, _TR), lambda bi, ri: (bi, 0, ri)),
            pl.BlockSpec((1, 1, n), lambda bi, ri: (bi, 0, 0)),
        ],
        out_specs=[
            pl.BlockSpec((1, _K, _TR), lambda bi, ri: (bi, 0, ri)),
            pl.BlockSpec((1, _K, _TR), lambda bi, ri: (bi, 0, ri)),
            pl.BlockSpec((1, 1, _TR), lambda bi, ri: (bi, 0, ri)),
        ],
        out_shape=[
            jax.ShapeDtypeStruct((b, _K, n), jnp.int32),
            jax.ShapeDtypeStruct((b, _K, n), jnp.float32),
            jax.ShapeDtypeStruct((b, 1, n), jnp.float32),
        ],
    )(xp, xpt, aa, aa)


def _sc_mesh():
    return plsc.VectorSubcoreMesh(core_axis_name="c", subcore_axis_name="s")


def _sc_gather(table, idx3):
    e, d = table.shape
    nch = idx3.shape[1]
    rw = nch * _CH
    r = _NW * rw

    @functools.partial(
        pl.kernel,
        out_type=jax.ShapeDtypeStruct((r, d), jnp.float32),
        mesh=_sc_mesh(),
        scratch_types=[
            pltpu.VMEM((nch, _CH), jnp.int32),
            pltpu.VMEM((rw, d), jnp.float32),
            pltpu.SemaphoreType.DMA,
        ],
        compiler_params=pltpu.CompilerParams(use_tc_tiling_on_sc=False),
    )
    def k(table_hbm, idx_hbm, out_hbm, idx_v, rows_v, sem):
        wid = lax.axis_index("s") * 2 + lax.axis_index("c")
        pltpu.sync_copy(idx_hbm.at[wid], idx_v)
        descs = []
        for c in range(nch):
            descs.append(pltpu.async_copy(
                table_hbm.at[idx_v.at[c]], rows_v.at[pl.ds(c * _CH, _CH)], sem))
        for de in descs:
            de.wait()
        pltpu.sync_copy(rows_v, out_hbm.at[pl.ds(wid * rw, rw)])

    return k(table, idx3)


def _sc_scatter_add(rows, idx3, e):
    r, d = rows.shape
    nch = idx3.shape[1]
    rw = nch * _CH
    npers = e // _NSUB
    zer = jnp.zeros((e, d), jnp.float32)

    @functools.partial(
        pl.kernel,
        out_type=jax.ShapeDtypeStruct((2, e, d), jnp.float32),
        mesh=_sc_mesh(),
        scratch_types=[
            pltpu.VMEM((nch, _CH), jnp.int32),
            pltpu.VMEM((rw, d), jnp.float32),
            pltpu.VMEM_SHARED((e, d), jnp.float32),
        ],
        compiler_params=pltpu.CompilerParams(use_tc_tiling_on_sc=False),
    )
    def k(rows_hbm, idx_hbm, zer_hbm, out_hbm, idx_v, rows_v, shared):
        cid = lax.axis_index("c")
        sid = lax.axis_index("s")
        wid = sid * 2 + cid
        pltpu.sync_copy(zer_hbm.at[pl.ds(sid * npers, npers)],
                        shared.at[pl.ds(sid * npers, npers)])
        pltpu.sync_copy(idx_hbm.at[wid], idx_v)
        pltpu.sync_copy(rows_hbm.at[pl.ds(wid * rw, rw)], rows_v)
        plsc.subcore_barrier()
        for c in range(nch):
            pltpu.sync_copy(rows_v.at[pl.ds(c * _CH, _CH)],
                            shared.at[idx_v.at[c]], add=True)
        plsc.subcore_barrier()
        pltpu.sync_copy(shared.at[pl.ds(sid * npers, npers)],
                        out_hbm.at[cid, pl.ds(sid * npers, npers)])

    return k(rows, idx3, zer)


def _combine_body(g_ref, w_ref, inv_ref, p_ref):
    t = (g_ref[0] * w_ref[0, 0][:, None]
         + g_ref[1] * w_ref[1, 0][:, None]
         + g_ref[2] * w_ref[2, 0][:, None])
    s = inv_ref[0, 0][:, None] * t
    for k in range(_K):
        p_ref[k] = w_ref[k, 0][:, None] * s


def _tc_combine(g, w3, invde3, e, d):
    te = 2048
    return pl.pallas_call(
        _combine_body,
        grid=(e // te,),
        in_specs=[
            pl.BlockSpec((_K, te, d), lambda i: (0, i, 0)),
            pl.BlockSpec((_K, 1, te), lambda i: (0, 0, i)),
            pl.BlockSpec((1, 1, te), lambda i: (0, 0, i)),
        ],
        out_specs=pl.BlockSpec((_K, te, d), lambda i: (0, i, 0)),
        out_shape=jax.ShapeDtypeStruct((_K, e, d), jnp.float32),
    )(g, w3, invde3)


def _final_body(y_ref, dv2_ref, wc_ref, bc_ref, out_ref):
    y = y_ref[0] + y_ref[1]
    h2 = dv2_ref[0, 0][:, None] * y
    f = jnp.max(h2, axis=0)
    out_ref[0, 0] = jnp.sum(wc_ref[...] * f[None, :], axis=1) + bc_ref[0]


def _tc_final(y, dv2_3, W_cls, b_cls, b, n, d):
    cls = W_cls.shape[0]
    out = pl.pallas_call(
        _final_body,
        grid=(b,),
        in_specs=[
            pl.BlockSpec((2, n, d), lambda bi: (0, bi, 0)),
            pl.BlockSpec((1, 1, n), lambda bi: (0, 0, bi)),
            pl.BlockSpec((cls, d), lambda bi: (0, 0)),
            pl.BlockSpec((1, cls), lambda bi: (0, 0)),
        ],
        out_specs=pl.BlockSpec((1, 1, cls), lambda bi: (bi, 0, 0)),
        out_shape=jax.ShapeDtypeStruct((b, 1, cls), jnp.float32),
    )(y, dv2_3, W_cls, b_cls.reshape(1, cls))
    return out.reshape(b, cls)


def kernel(x, theta, bias, W_cls, b_cls):
    b, _, n, _ = x.shape
    hid = theta.shape[1]
    e = b * n
    r = e * _K
    nch = r // (_NW * _CH)
    xl = x[:, -1]
    idx, w, invde = _knn_topk(xl)

    gidx = (idx + (jnp.arange(b, dtype=jnp.int32) * n)[:, None, None])
    gidx_t = gidx.transpose(1, 0, 2).reshape(_K, e)
    idx3 = gidx_t.reshape(_NW, nch, _CH)
    wt = w.transpose(1, 0, 2).reshape(_K, e)
    w3 = wt.reshape(_K, 1, e)
    invde3 = invde.reshape(1, 1, e)

    p_dv = jnp.broadcast_to(wt.reshape(r)[:, None], (r, 16))
    dv_part = _sc_scatter_add(p_dv, idx3, e)
    dv = dv_part[0, :, 0] + dv_part[1, :, 0]
    dv2 = jnp.where(dv > 0, dv ** -0.5, 0.0)

    theta_t = jnp.broadcast_to(theta[None], (b, n, hid)).reshape(e, hid)

    def apply_op(u):
        g = _sc_gather(u, idx3).reshape(_K, e, hid)
        p = _tc_combine(g, w3, invde3, e, hid).reshape(r, hid)
        return _sc_scatter_add(p, idx3, e)

    y0 = apply_op(dv2[:, None] * theta_t)
    h1 = dv2[:, None] * (y0[0] + y0[1]) + bias
    y1 = apply_op(dv2[:, None] * h1)
    return _tc_final(y1, dv2.reshape(1, 1, e), W_cls, b_cls, b, n, hid)


# speedup vs baseline: 11.1237x; 1.0321x over previous
import functools

import jax
import jax.numpy as jnp
from jax import lax
from jax.experimental import pallas as pl
from jax.experimental.pallas import tpu as pltpu
from jax.experimental.pallas import tpu_sc as plsc

_K = 3
_TR = 512
_NW = 32
_NSUB = 16
_CH = 128


def _knn_body(xi_ref, xt_ref, aai_ref, aaj_ref, iout_ref, wout_ref, dout_ref, *, n):
    xi = xi_ref[0]
    xt = xt_ref[0]
    aa_i = aai_ref[0, 0]
    aa_j = aaj_ref[0, 0]
    dotp = jnp.dot(xi, xt, preferred_element_type=jnp.float32)
    d2 = aa_i[:, None] + aa_j[None, :] - 2.0 * dotp
    d2 = jnp.maximum(d2, 0.0)
    dis = jnp.sqrt(d2)
    avg = jnp.mean(dis, axis=1)
    denom = avg * avg + 1e-12
    iota = jax.lax.broadcasted_iota(jnp.int32, (_TR, n), 1).astype(jnp.float32)
    cur = dis
    ws = []
    for k in range(_K):
        m = jnp.min(cur, axis=1)
        eq = cur == m[:, None]
        ikf = jnp.min(jnp.where(eq, iota, jnp.float32(n)), axis=1)
        wk = jnp.exp(-(m * m) / denom)
        iout_ref[0, k, :] = ikf.astype(jnp.int32)
        wout_ref[0, k, :] = wk
        ws.append(wk)
        if k < _K - 1:
            cur = jnp.where(iota == ikf[:, None], jnp.inf, cur)
    dout_ref[0, 0, :] = 1.0 / (ws[0] + ws[1] + ws[2])


def _knn_topk(xl):
    b, n, _ = xl.shape
    xp = jnp.pad(xl, ((0, 0), (0, 0), (0, 5)))
    xpt = xp.transpose(0, 2, 1)
    aa = jnp.sum(xl * xl, axis=2).reshape(b, 1, n)
    grid = (b, n // _TR)
    return pl.pallas_call(
        functools.partial(_knn_body, n=n),
        grid=grid,
        in_specs=[
            pl.BlockSpec((1, _TR, 8), lambda bi, ri: (bi, ri, 0)),
            pl.BlockSpec((1, 8, n), lambda bi, ri: (bi, 0, 0)),
            pl.BlockSpec((1, 1, _TR), lambda bi, ri: (bi, 0, ri)),
            pl.BlockSpec((1, 1, n), lambda bi, ri: (bi, 0, 0)),
        ],
        out_specs=[
            pl.BlockSpec((1, _K, _TR), lambda bi, ri: (bi, 0, ri)),
            pl.BlockSpec((1, _K, _TR), lambda bi, ri: (bi, 0, ri)),
            pl.BlockSpec((1, 1, _TR), lambda bi, ri: (bi, 0, ri)),
        ],
        out_shape=[
            jax.ShapeDtypeStruct((b, _K, n), jnp.int32),
            jax.ShapeDtypeStruct((b, _K, n), jnp.float32),
            jax.ShapeDtypeStruct((b, 1, n), jnp.float32),
        ],
    )(xp, xpt, aa, aa)


def _sc_mesh():
    return plsc.VectorSubcoreMesh(core_axis_name="c", subcore_axis_name="s")


def _sc_gather(table, idx3):
    e, d = table.shape
    nch = idx3.shape[1]
    rw = nch * _CH
    r = _NW * rw

    @functools.partial(
        pl.kernel,
        out_type=jax.ShapeDtypeStruct((r, d), jnp.float32),
        mesh=_sc_mesh(),
        scratch_types=[
            pltpu.VMEM((nch, _CH), jnp.int32),
            pltpu.VMEM((rw, d), jnp.float32),
            pltpu.SemaphoreType.DMA,
        ],
        compiler_params=pltpu.CompilerParams(use_tc_tiling_on_sc=False),
    )
    def k(table_hbm, idx_hbm, out_hbm, idx_v, rows_v, sem):
        wid = lax.axis_index("s") * 2 + lax.axis_index("c")
        pltpu.sync_copy(idx_hbm.at[wid], idx_v)
        descs = []
        for c in range(nch):
            descs.append(pltpu.async_copy(
                table_hbm.at[idx_v.at[c]], rows_v.at[pl.ds(c * _CH, _CH)], sem))
        for de in descs:
            de.wait()
        pltpu.sync_copy(rows_v, out_hbm.at[pl.ds(wid * rw, rw)])

    return k(table, idx3)


def _sc_scatter_add(rows, idx3, e):
    r, d = rows.shape
    nch = idx3.shape[1]
    rw = nch * _CH
    npers = e // _NSUB
    zer = jnp.zeros((e, d), jnp.float32)

    @functools.partial(
        pl.kernel,
        out_type=jax.ShapeDtypeStruct((2, e, d), jnp.float32),
        mesh=_sc_mesh(),
        scratch_types=[
            pltpu.VMEM((nch, _CH), jnp.int32),
            pltpu.VMEM((rw, d), jnp.float32),
            pltpu.VMEM_SHARED((e, d), jnp.float32),
        ],
        compiler_params=pltpu.CompilerParams(use_tc_tiling_on_sc=False),
    )
    def k(rows_hbm, idx_hbm, zer_hbm, out_hbm, idx_v, rows_v, shared):
        cid = lax.axis_index("c")
        sid = lax.axis_index("s")
        wid = sid * 2 + cid
        pltpu.sync_copy(zer_hbm.at[pl.ds(sid * npers, npers)],
                        shared.at[pl.ds(sid * npers, npers)])
        pltpu.sync_copy(idx_hbm.at[wid], idx_v)
        pltpu.sync_copy(rows_hbm.at[pl.ds(wid * rw, rw)], rows_v)
        plsc.subcore_barrier()
        for c in range(nch):
            pltpu.sync_copy(rows_v.at[pl.ds(c * _CH, _CH)],
                            shared.at[idx_v.at[c]], add=True)
        plsc.subcore_barrier()
        pltpu.sync_copy(shared.at[pl.ds(sid * npers, npers)],
                        out_hbm.at[cid, pl.ds(sid * npers, npers)])

    return k(rows, idx3, zer)


def _combine_body(g_ref, w_ref, inv_ref, p_ref):
    t = (g_ref[0] * w_ref[0, 0][:, None]
         + g_ref[1] * w_ref[1, 0][:, None]
         + g_ref[2] * w_ref[2, 0][:, None])
    s = inv_ref[0, 0][:, None] * t
    for k in range(_K):
        p_ref[k] = w_ref[k, 0][:, None] * s


def _tc_combine(g, w3, invde3, e, d):
    te = 2048
    return pl.pallas_call(
        _combine_body,
        grid=(e // te,),
        in_specs=[
            pl.BlockSpec((_K, te, d), lambda i: (0, i, 0)),
            pl.BlockSpec((_K, 1, te), lambda i: (0, 0, i)),
            pl.BlockSpec((1, 1, te), lambda i: (0, 0, i)),
        ],
        out_specs=pl.BlockSpec((_K, te, d), lambda i: (0, i, 0)),
        out_shape=jax.ShapeDtypeStruct((_K, e, d), jnp.float32),
    )(g, w3, invde3)


def _final_body(y_ref, dv2_ref, wc_ref, bc_ref, out_ref):
    y = y_ref[0] + y_ref[1]
    h2 = dv2_ref[0, 0][:, None] * y
    f = jnp.max(h2, axis=0)
    out_ref[0, 0] = jnp.sum(wc_ref[...] * f[None, :], axis=1) + bc_ref[0]


def _tc_final(y, dv2_3, W_cls, b_cls, b, n, d):
    cls = W_cls.shape[0]
    out = pl.pallas_call(
        _final_body,
        grid=(b,),
        in_specs=[
            pl.BlockSpec((2, n, d), lambda bi: (0, bi, 0)),
            pl.BlockSpec((1, 1, n), lambda bi: (0, 0, bi)),
            pl.BlockSpec((cls, d), lambda bi: (0, 0)),
            pl.BlockSpec((1, cls), lambda bi: (0, 0)),
        ],
        out_specs=pl.BlockSpec((1, 1, cls), lambda bi: (bi, 0, 0)),
        out_shape=jax.ShapeDtypeStruct((b, 1, cls), jnp.float32),
    )(y, dv2_3, W_cls, b_cls.reshape(1, cls))
    return out.reshape(b, cls)


def kernel(x, theta, bias, W_cls, b_cls):
    b, _, n, _ = x.shape
    hid = theta.shape[1]
    e = b * n
    r = e * _K
    nch = r // (_NW * _CH)
    xl = x[:, -1]
    idx, w, invde = _knn_topk(xl)

    gidx = (idx + (jnp.arange(b, dtype=jnp.int32) * n)[:, None, None])
    gidx_t = gidx.transpose(1, 0, 2).reshape(_K, e)
    idx3 = gidx_t.reshape(_NW, nch, _CH)
    wt = w.transpose(1, 0, 2).reshape(_K, e)
    w3 = wt.reshape(_K, 1, e)
    invde3 = invde.reshape(1, 1, e)

    p_dv = jnp.broadcast_to(wt.reshape(r)[:, None], (r, 16))
    dv_part = _sc_scatter_add(p_dv, idx3, e)
    dv = dv_part[0, :, 0] + dv_part[1, :, 0]
    dv2 = jnp.where(dv > 0, dv ** -0.5, 0.0)

    theta_t = jnp.broadcast_to(theta[None], (b, n, hid)).reshape(e, hid)

    def apply_op(u):
        g = _sc_gather(u, idx3).reshape(_K, e, hid)
        p = _tc_combine(g, w3, invde3, e, hid).reshape(r, hid)
        return _sc_scatter_add(p, idx3, e)

    y0 = apply_op(dv2[:, None] * theta_t)
    h1 = dv2[:, None] * (y0[0] + y0[1]) + bias
    y1 = apply_op(dv2[:, None] * h1)
    return _tc_final(y1, dv2.reshape(1, 1, e), W_cls, b_cls, b, n, hid)
